# baseline (speedup 1.0000x reference)
import functools

import jax
import jax.numpy as jnp
from jax import lax
from jax.experimental import pallas as pl
from jax.experimental.pallas import tpu as pltpu
from jax.experimental.pallas import tpu_sc as plsc

_CHUNK = 128


def kernel(indices, weight):
    (B,) = indices.shape
    V, D = weight.shape
    info = plsc.get_sparse_core_info()
    num_workers = info.num_cores * info.num_subcores
    n_chunks = B // _CHUNK
    k = n_chunks // num_workers

    idx2d = indices.astype(jnp.int32).reshape(n_chunks, _CHUNK)
    mesh = plsc.VectorSubcoreMesh(core_axis_name="c", subcore_axis_name="s")

    @functools.partial(
        pl.kernel,
        mesh=mesh,
        out_type=jax.ShapeDtypeStruct((n_chunks, _CHUNK, D), jnp.float32),
        scratch_types=[
            pltpu.VMEM((k, _CHUNK), jnp.int32),
            pltpu.VMEM((k, _CHUNK, D), jnp.float32),
            pltpu.SemaphoreType.DMA,
        ],
        compiler_params=pltpu.CompilerParams(use_tc_tiling_on_sc=False),
    )
    def gather_kernel(idx_hbm, table_hbm, out_hbm, idx_v, rows_v, sem):
        wid = lax.axis_index("s") * info.num_cores + lax.axis_index("c")
        base = wid * k
        pltpu.sync_copy(idx_hbm.at[pl.ds(base, k)], idx_v)
        copies = [
            pltpu.async_copy(table_hbm.at[idx_v.at[j]], rows_v.at[j], sem)
            for j in range(k)
        ]
        for c in copies:
            c.wait()
        pltpu.sync_copy(rows_v, out_hbm.at[pl.ds(base, k)])

    out = gather_kernel(idx2d, weight)
    return out.reshape(B, D)


# speedup vs baseline: 1.0276x; 1.0276x over previous
import functools

import jax
import jax.numpy as jnp
from jax import lax
from jax.experimental import pallas as pl
from jax.experimental.pallas import tpu as pltpu
from jax.experimental.pallas import tpu_sc as plsc

_L = 16


def kernel(indices, weight):
    (B,) = indices.shape
    V, D = weight.shape
    info = plsc.get_sparse_core_info()
    num_workers = info.num_cores * info.num_subcores
    per = B // num_workers

    idx32 = indices.astype(jnp.int32)
    mesh = plsc.VectorSubcoreMesh(core_axis_name="c", subcore_axis_name="s")

    @functools.partial(
        pl.kernel,
        mesh=mesh,
        out_type=jax.ShapeDtypeStruct((B, D), jnp.float32),
        scratch_types=[
            pltpu.VMEM((per,), jnp.int32),
            pltpu.SemaphoreType.DMA,
        ],
    )
    def gather_kernel(idx_hbm, table_hbm, out_hbm, idx_v, sem):
        wid = lax.axis_index("s") * info.num_cores + lax.axis_index("c")
        base = wid * per
        pltpu.sync_copy(idx_hbm.at[pl.ds(base, per)], idx_v)

        def body(g, carry):
            vec = idx_v[pl.ds(g * _L, _L)]
            for j in range(_L):
                row = vec[j]
                pltpu.async_copy(
                    table_hbm.at[pl.ds(row, 1)],
                    out_hbm.at[pl.ds(base + g * _L + j, 1)],
                    sem,
                )
            return carry

        lax.fori_loop(0, per // _L, body, 0)
        pltpu.make_async_copy(
            table_hbm.at[pl.ds(0, per)], out_hbm.at[pl.ds(base, per)], sem
        ).wait()

    return gather_kernel(idx32, weight)


# speedup vs baseline: 1.7137x; 1.6676x over previous
import functools

import jax
import jax.numpy as jnp
from jax import lax
from jax.experimental import pallas as pl
from jax.experimental.pallas import tpu as pltpu
from jax.experimental.pallas import tpu_sc as plsc

_L = 16


def kernel(indices, weight):
    (B,) = indices.shape
    V, D = weight.shape
    info = plsc.get_sparse_core_info()
    num_workers = info.num_cores * info.num_subcores
    per = B // num_workers

    idx32 = indices.astype(jnp.int32)
    mesh = plsc.VectorSubcoreMesh(core_axis_name="c", subcore_axis_name="s")

    @functools.partial(
        pl.kernel,
        mesh=mesh,
        out_type=jax.ShapeDtypeStruct((B, D), jnp.float32),
        scratch_types=[
            pltpu.VMEM((per,), jnp.int32),
            pltpu.VMEM((per, 64), jnp.float32),
            pltpu.SemaphoreType.DMA,
        ],
    )
    def gather_kernel(idx_hbm, table_hbm, out_hbm, idx_v, rows_v, sem):
        wid = lax.axis_index("s") * info.num_cores + lax.axis_index("c")
        base = wid * per
        pltpu.sync_copy(idx_hbm.at[pl.ds(base, per)], idx_v)

        def body(g, carry):
            vec = idx_v[pl.ds(g * _L, _L)]
            for j in range(_L):
                row = vec[j]
                pltpu.async_copy(
                    table_hbm.at[pl.ds(row, 1)],
                    rows_v.at[pl.ds(g * _L + j, 1)],
                    sem,
                )
            return carry

        lax.fori_loop(0, per // _L, body, 0)
        pltpu.make_async_copy(
            table_hbm.at[pl.ds(0, per)], rows_v, sem
        ).wait()
        pltpu.sync_copy(rows_v, out_hbm.at[pl.ds(base, per)])

    return gather_kernel(idx32, weight)
